# Initial kernel scaffold; baseline (speedup 1.0000x reference)
import jax
import jax.numpy as jnp
from jax.experimental import pallas as pl


def kernel(x, edge_index, W1, b1, g1, be1, W2, b2, g2, be2, W3, b3, g3, be3):
    raise NotImplementedError("write your pallas kernel here")


# speedup vs baseline: 23.6085x; 23.6085x over previous
import functools

import jax
import jax.numpy as jnp
from jax import lax
from jax.experimental import pallas as pl
from jax.experimental.pallas import tpu as pltpu
from jax.experimental.pallas import tpu_sc as plsc

_NC = 2
_NS = 16
_NW = _NC * _NS


def _pick_chunk(epw):
    for c in range(128, 7, -1):
        if c % 8 == 0 and epw % c == 0:
            return c
    raise ValueError(epw)


def _stripe(n):
    rps = (n // _NS) & ~7
    tail = n - _NS * rps
    return rps, tail


def _init_stripes(src_hbm, dst_sh, si, rps, tail, n):
    off = pl.multiple_of(si * rps, 8)
    pltpu.sync_copy(src_hbm.at[pl.ds(off, rps)], dst_sh.at[pl.ds(off, rps)])
    if tail:
        @pl.when(si == 0)
        def _():
            pltpu.sync_copy(src_hbm.at[pl.ds(_NS * rps, tail)],
                            dst_sh.at[pl.ds(_NS * rps, tail)])


def _drain_stripes(src_sh, out_hbm, ci, si, rps, tail, n):
    off = pl.multiple_of(si * rps, 8)
    obase = pl.multiple_of(ci * n, 8)
    pltpu.sync_copy(src_sh.at[pl.ds(off, rps)],
                    out_hbm.at[pl.ds(obase + off, rps)])
    if tail:
        @pl.when(si == 0)
        def _():
            pltpu.sync_copy(src_sh.at[pl.ds(_NS * rps, tail)],
                            out_hbm.at[pl.ds(obase + _NS * rps, tail)])


@functools.partial(jax.jit, static_argnames=("n", "d", "e"))
def _sc_degree(dst3, zerosnd, onesc, *, n, d, e):
    epw = e // _NW
    c = _pick_chunk(epw)
    iters = epw // c
    rps, tail = _stripe(n)
    mesh = plsc.VectorSubcoreMesh(core_axis_name="c", subcore_axis_name="s")

    @functools.partial(
        pl.kernel,
        mesh=mesh,
        out_type=jax.ShapeDtypeStruct((2 * n, d), jnp.float32),
        scratch_types=[
            pltpu.VMEM((iters, c), jnp.int32),
            pltpu.VMEM((c, d), jnp.float32),
            pltpu.VMEM_SHARED((n, d), jnp.float32),
        ],
    )
    def k(dst_hbm, z_hbm, ones_hbm, out_hbm, idx_v, ones_v, acc_sh):
        ci = lax.axis_index("c")
        si = lax.axis_index("s")
        wid = si * _NC + ci
        pltpu.sync_copy(ones_hbm, ones_v)
        pltpu.sync_copy(dst_hbm.at[wid], idx_v)
        _init_stripes(z_hbm, acc_sh, si, rps, tail, n)
        plsc.subcore_barrier()

        def body(j, carry):
            pltpu.sync_copy(ones_v, acc_sh.at[idx_v.at[j]], add=True)
            return carry

        lax.fori_loop(0, iters, body, 0)
        plsc.subcore_barrier()
        _drain_stripes(acc_sh, out_hbm, ci, si, rps, tail, n)

    return k(dst3, zerosnd, onesc)


@functools.partial(jax.jit, static_argnames=("n", "d", "e"))
def _sc_propagate(zs, src, dst3, zerosnd, *, n, d, e):
    epw = e // _NW
    c = _pick_chunk(epw)
    iters = epw // c
    rps, tail = _stripe(n)
    mesh = plsc.VectorSubcoreMesh(core_axis_name="c", subcore_axis_name="s")

    @functools.partial(
        pl.kernel,
        mesh=mesh,
        out_type=jax.ShapeDtypeStruct((2 * n, d), jnp.float32),
        scratch_types=[
            pltpu.VMEM((epw,), jnp.int32),
            pltpu.VMEM((iters, c), jnp.int32),
            pltpu.VMEM((c, d), jnp.float32),
            pltpu.VMEM((c, d), jnp.float32),
            pltpu.VMEM_SHARED((n, d), jnp.float32),
            pltpu.SemaphoreType.DMA,
            pltpu.SemaphoreType.DMA,
        ],
    )
    def k(zs_hbm, src_hbm, dst_hbm, z_hbm, out_hbm,
          sidx_v, didx_v, rows0, rows1, acc_sh, sem0, sem1):
        ci = lax.axis_index("c")
        si = lax.axis_index("s")
        wid = si * _NC + ci
        pltpu.sync_copy(src_hbm.at[pl.ds(pl.multiple_of(wid * epw, 8), epw)],
                        sidx_v)
        pltpu.sync_copy(dst_hbm.at[wid], didx_v)
        _init_stripes(z_hbm, acc_sh, si, rps, tail, n)
        plsc.subcore_barrier()

        def gather(j, rows, sem):
            pltpu.async_copy(zs_hbm.at[sidx_v.at[pl.ds(j * c, c)]], rows, sem)

        def wait_gather(j, rows, sem):
            pltpu.make_async_copy(zs_hbm.at[sidx_v.at[pl.ds(j * c, c)]],
                                  rows, sem).wait()

        def scatter(j, rows):
            pltpu.sync_copy(rows, acc_sh.at[didx_v.at[j]], add=True)

        gather(0, rows0, sem0)

        def body(t, carry):
            j0 = 2 * t
            gather(j0 + 1, rows1, sem1)
            wait_gather(j0, rows0, sem0)
            scatter(j0, rows0)
            gather(j0 + 2, rows0, sem0)
            wait_gather(j0 + 1, rows1, sem1)
            scatter(j0 + 1, rows1)
            return carry

        pairs = (iters - 1) // 2
        lax.fori_loop(0, pairs, body, 0)
        if iters % 2 == 0:
            gather(iters - 1, rows1, sem1)
            wait_gather(iters - 2, rows0, sem0)
            scatter(iters - 2, rows0)
            wait_gather(iters - 1, rows1, sem1)
            scatter(iters - 1, rows1)
        else:
            wait_gather(iters - 1, rows0, sem0)
            scatter(iters - 1, rows0)
        plsc.subcore_barrier()
        _drain_stripes(acc_sh, out_hbm, ci, si, rps, tail, n)

    return k(zs, src, dst3, zerosnd)


def _tc0_body(dp_ref, x_ref, w_ref, dis_ref, zs_ref):
    n = x_ref.shape[0]
    dp = dp_ref[...]
    deg = dp[:n, 0:1] + dp[n:, 0:1] + 1.0
    dis = lax.rsqrt(deg)
    dis_full = jnp.broadcast_to(dis, zs_ref.shape)
    dis_ref[...] = dis_full
    zs_ref[...] = dis_full * jnp.dot(x_ref[...], w_ref[...],
                                     preferred_element_type=jnp.float32)


def _tc0(deg_parts, x, w):
    n, dd = x.shape[0], w.shape[1]
    return pl.pallas_call(
        _tc0_body,
        out_shape=(jax.ShapeDtypeStruct((n, dd), jnp.float32),
                   jax.ShapeDtypeStruct((n, dd), jnp.float32)),
    )(deg_parts, x, w)


def _bn_input(p_ref, zs_ref, dis_ref, b_ref):
    n = zs_ref.shape[0]
    p = p_ref[...]
    return dis_ref[...] * (p[:n] + p[n:] + zs_ref[...]) + b_ref[...]


def _bn(h, g_ref, be_ref):
    mu = jnp.mean(h, axis=0, keepdims=True)
    var = jnp.mean((h - mu) * (h - mu), axis=0, keepdims=True)
    return g_ref[...] * (h - mu) * lax.rsqrt(var + 1e-5) + be_ref[...]


def _tcmid_body(p_ref, zs_ref, dis_ref, b_ref, g_ref, be_ref, w_ref, out_ref):
    h = _bn_input(p_ref, zs_ref, dis_ref, b_ref)
    hr = jnp.maximum(_bn(h, g_ref, be_ref), 0.0)
    out_ref[...] = dis_ref[...] * jnp.dot(hr, w_ref[...],
                                          preferred_element_type=jnp.float32)


def _tcmid(parts, zs, dis, b, g, be, w):
    n, dd = zs.shape
    return pl.pallas_call(
        _tcmid_body,
        out_shape=jax.ShapeDtypeStruct((n, w.shape[1]), jnp.float32),
    )(parts, zs, dis, b.reshape(1, dd), g.reshape(1, dd), be.reshape(1, dd), w)


def _tclast_body(p_ref, zs_ref, dis_ref, b_ref, g_ref, be_ref, out_ref):
    h = _bn_input(p_ref, zs_ref, dis_ref, b_ref)
    h = _bn(h, g_ref, be_ref)
    m = jnp.max(h, axis=1, keepdims=True)
    lse = jnp.log(jnp.sum(jnp.exp(h - m), axis=1, keepdims=True)) + m
    out_ref[...] = h - lse


def _tclast(parts, zs, dis, b, g, be):
    n, dd = zs.shape
    return pl.pallas_call(
        _tclast_body,
        out_shape=jax.ShapeDtypeStruct((n, dd), jnp.float32),
    )(parts, zs, dis, b.reshape(1, dd), g.reshape(1, dd), be.reshape(1, dd))


def kernel(x, edge_index, W1, b1, g1, be1, W2, b2, g2, be2, W3, b3, g3, be3):
    n, din = x.shape
    e = edge_index.shape[1]
    d = W1.shape[1]
    src = edge_index[0]
    dst = edge_index[1]
    epw = e // _NW
    c = _pick_chunk(epw)
    iters = epw // c

    zerosnd = jnp.zeros((n, d), jnp.float32)
    onesc = jnp.ones((c, d), jnp.float32)
    dst3 = dst.reshape(_NW, iters, c)

    deg_parts = _sc_degree(dst3, zerosnd, onesc, n=n, d=d, e=e)
    dis, zs1 = _tc0(deg_parts, x, W1)
    p1 = _sc_propagate(zs1, src, dst3, zerosnd, n=n, d=d, e=e)
    zs2 = _tcmid(p1, zs1, dis, b1, g1, be1, W2)
    p2 = _sc_propagate(zs2, src, dst3, zerosnd, n=n, d=d, e=e)
    zs3 = _tcmid(p2, zs2, dis, b2, g2, be2, W3)
    p3 = _sc_propagate(zs3, src, dst3, zerosnd, n=n, d=d, e=e)
    return _tclast(p3, zs3, dis, b3, g3, be3)
